# baseline (speedup 1.0000x reference)
import jax
import jax.numpy as jnp
from jax.experimental import pallas as pl
from jax.experimental.pallas import tpu as pltpu

_VMEM_LIMIT = 48 * 1024 * 1024


def _pc(i, d):
    return i % d, i // d + 1


def _halo_store(ref, interior, dtype=None):
    h, w, c = ref.shape
    ref[1:h - 1, 1:w - 1, :] = interior.astype(ref.dtype)


def _im2col9_single(x, H, W, C):
    cols = [x[:, kx:kx + W, :] for kx in range(3)]
    return jnp.concatenate(
        [cols[kx][ky:ky + H].reshape(H * W, C)
         for kx in range(3) for ky in range(3)],
        axis=-1)


def _plane_tap(planes, d, arow, acol, u, v, P, C):
    r, dr = _pc(arow + u - 1, d)
    c, dc = _pc(acol + v - 1, d)
    return planes[r * d + c][dr:dr + P, dc:dc + P, :].reshape(P * P, C)


def _im2col9_planes(planes, d, arow, acol, P, C):
    return jnp.concatenate(
        [_plane_tap(planes, d, arow, acol, u, v, P, C)
         for v in range(3) for u in range(3)],
        axis=-1)


def _ct_lhs(planes, d, arow, acol, py, P, C):
    pieces = []
    for c in range(3):
        cc, dc = _pc(acol + c - 1, d)
        for dy in range(2):
            rr, dr = _pc(arow + py + dy - 1, d)
            pieces.append(
                planes[rr * d + cc][dr:dr + P, dc:dc + P, :].reshape(
                    P * P, C))
    return jnp.concatenate(pieces, axis=-1)


def _res_block_single(xp, w9, b3, w1, b1, P, C):
    HW = P * P
    xr = jnp.maximum(xp, 0).astype(jnp.bfloat16)
    x9 = _im2col9_single(xr, P, P, C)
    acc = jnp.dot(x9, w9, preferred_element_type=jnp.float32)
    h = jnp.maximum(acc + b3, 0.0).astype(jnp.bfloat16)
    out = jnp.dot(h, w1, preferred_element_type=jnp.float32) + b1
    skip = xp[1:1 + P, 1:1 + P, :].reshape(HW, C).astype(jnp.float32)
    return out + skip


def _res_pair_planes(pin_refs, ptmp_refs, w9_ref, b3_ref, w1_ref, b1_ref,
                     P, C):
    HW = P * P
    n = len(pin_refs)
    d = 2 if n == 4 else 1
    for src, dst, blk in ((pin_refs, ptmp_refs, 0), (ptmp_refs, pin_refs, 1)):
        vals = [src[i][...] for i in range(n)]
        xr = [jnp.maximum(v, 0).astype(jnp.bfloat16) for v in vals]
        x9 = jnp.concatenate(
            [_im2col9_planes(xr, d, a, b, P, C)
             for a in range(d) for b in range(d)], axis=0)
        acc = jnp.dot(x9, w9_ref[blk], preferred_element_type=jnp.float32)
        h = jnp.maximum(acc + b3_ref[blk], 0.0).astype(jnp.bfloat16)
        out = jnp.dot(h, w1_ref[blk],
                      preferred_element_type=jnp.float32) + b1_ref[blk]
        skip = jnp.concatenate(
            [vals[i][1:1 + P, 1:1 + P, :].reshape(HW, C)
             for i in range(n)], axis=0).astype(jnp.float32)
        res = out + skip
        for i in range(n):
            _halo_store(dst[i], res[i * HW:(i + 1) * HW].reshape(P, P, C))


def _decoder_body(P, C):
    HW = P * P

    def body(x0_ref, y_ref,
             w9a_ref, b3a_ref, w1a_ref, b1a_ref,
             wct1_ref, bct1_ref,
             wc2_ref, bc2_ref,
             w9b_ref, b3b_ref, w1b_ref, b1b_ref,
             wct3_ref, bct3_ref,
             wct4_ref, bct4_ref,
             o_ref,
             m16_ref, pa_ref, pb_ref, pc_ref, p64_ref):
        pa = [pa_ref.at[i] for i in range(4)]
        pb = [pb_ref.at[i] for i in range(4)]
        pc = [pc_ref.at[i] for i in range(4)]
        p64 = [p64_ref.at[i] for i in range(16)]

        @pl.when(pl.program_id(0) == 0)
        def _init_borders():
            m16_ref[...] = jnp.zeros(m16_ref.shape, m16_ref.dtype)
            pa_ref[...] = jnp.zeros(pa_ref.shape, pa_ref.dtype)
            pb_ref[...] = jnp.zeros(pb_ref.shape, pb_ref.dtype)
            pc_ref[...] = jnp.zeros(pc_ref.shape, pc_ref.dtype)
            p64_ref[...] = jnp.zeros(p64_ref.shape, p64_ref.dtype)

        x1 = _res_block_single(x0_ref[0], w9a_ref[0], b3a_ref[0],
                               w1a_ref[0], b1a_ref[0], P, C)
        _halo_store(m16_ref, x1.reshape(P, P, C))
        x2 = _res_block_single(m16_ref[...], w9a_ref[1], b3a_ref[1],
                               w1a_ref[1], b1a_ref[1], P, C)
        _halo_store(m16_ref, x2.reshape(P, P, C))

        x = jnp.maximum(m16_ref[...], 0).astype(jnp.bfloat16)
        cols = [x[:, c:c + P, :] for c in range(3)]
        s = [[cols[c][r:r + P].reshape(HW, C) for r in range(3)]
             for c in range(3)]
        for py in range(2):
            lhs = jnp.concatenate(
                [s[c][py + dy] for c in range(3) for dy in range(2)], axis=-1)
            acc = jnp.dot(lhs, wct1_ref[py],
                          preferred_element_type=jnp.float32) + bct1_ref[...]
            acc = jnp.maximum(acc, 0.0)
            _halo_store(pa[2 * py], acc[:, :C].reshape(P, P, C))
            _halo_store(pa[2 * py + 1], acc[:, C:].reshape(P, P, C))

        avals = [pa_ref[i] for i in range(4)]
        yvals = [y_ref[0, i] for i in range(4)]
        rows = []
        for a in range(2):
            for b in range(2):
                pieces = []
                for v in range(3):
                    for u in range(3):
                        pieces.append(
                            _plane_tap(avals, 2, a, b, u, v, P, C))
                        pieces.append(
                            _plane_tap(yvals, 2, a, b, u, v, P, C))
                rows.append(jnp.concatenate(pieces, axis=-1))
        x18 = jnp.concatenate(rows, axis=0)
        out = jnp.dot(x18, wc2_ref[...],
                      preferred_element_type=jnp.float32) + bc2_ref[...]
        for i in range(4):
            _halo_store(pb[i], out[i * HW:(i + 1) * HW].reshape(P, P, C))

        _res_pair_planes(pb, pc, w9b_ref, b3b_ref, w1b_ref, b1b_ref, P, C)

        xr = [jnp.maximum(pb_ref[i], 0).astype(jnp.bfloat16) for i in range(4)]
        for py in range(2):
            lhs = jnp.concatenate(
                [_ct_lhs(xr, 2, a, b, py, P, C)
                 for a in range(2) for b in range(2)], axis=0)
            acc = jnp.dot(lhs, wct3_ref[py],
                          preferred_element_type=jnp.float32)
            acc = jnp.maximum(acc + bct3_ref[...], 0.0)
            for a in range(2):
                for b in range(2):
                    sub = acc[(a * 2 + b) * HW:(a * 2 + b + 1) * HW]
                    q = 2 * a + py
                    _halo_store(p64[q * 4 + 2 * b],
                                sub[:, :C].reshape(P, P, C))
                    _halo_store(p64[q * 4 + 2 * b + 1],
                                sub[:, C:].reshape(P, P, C))

        xv = [p64_ref[i] for i in range(16)]
        for py in range(2):
            lhs = jnp.concatenate(
                [_ct_lhs(xv, 4, q, sidx, py, P, C)
                 for q in range(4) for sidx in range(4)], axis=0)
            acc = jnp.dot(lhs, wct4_ref[py],
                          preferred_element_type=jnp.float32) + bct4_ref[...]
            for q in range(4):
                for sidx in range(4):
                    sub = acc[(q * 4 + sidx) * HW:(q * 4 + sidx + 1) * HW]
                    r8 = 2 * q + py
                    o_ref[0, r8, 2 * sidx] = sub[:, :3].reshape(P, P, 3)
                    o_ref[0, r8, 2 * sidx + 1] = sub[:, C:C + 3].reshape(
                        P, P, 3)

    return body


def _const_spec(*shape):
    nz = len(shape)
    return pl.BlockSpec(shape, lambda b, _n=nz: (0,) * _n)


def _decoder(x0p, yplanes, packed):
    B = x0p.shape[0]
    C = x0p.shape[-1]
    P = x0p.shape[1] - 2
    (w9a, b3a, w1a, b1a, wct1, bct1, wc2, bc2,
     w9b, b3b, w1b, b1b, wct3, bct3, wct4, bct4) = packed
    halo = P + 2
    return pl.pallas_call(
        _decoder_body(P, C),
        out_shape=jax.ShapeDtypeStruct((B, 8, 8, P, P, 3), jnp.float32),
        grid=(B,),
        in_specs=[
            pl.BlockSpec((1, halo, halo, C), lambda b: (b, 0, 0, 0)),
            pl.BlockSpec((1, 4, halo, halo, C), lambda b: (b, 0, 0, 0, 0)),
            _const_spec(2, 9 * C, C), _const_spec(2, 1, C),
            _const_spec(2, C, C), _const_spec(2, 1, C),
            _const_spec(2, 6 * C, 2 * C), _const_spec(1, 2 * C),
            _const_spec(18 * C, C), _const_spec(1, C),
            _const_spec(2, 9 * C, C), _const_spec(2, 1, C),
            _const_spec(2, C, C), _const_spec(2, 1, C),
            _const_spec(2, 6 * C, 2 * C), _const_spec(1, 2 * C),
            _const_spec(2, 6 * C, 2 * C), _const_spec(1, 2 * C),
        ],
        out_specs=pl.BlockSpec((1, 8, 8, P, P, 3),
                               lambda b: (b, 0, 0, 0, 0, 0)),
        scratch_shapes=[
            pltpu.VMEM((halo, halo, C), jnp.bfloat16),
            pltpu.VMEM((4, halo, halo, C), jnp.bfloat16),
            pltpu.VMEM((4, halo, halo, C), jnp.bfloat16),
            pltpu.VMEM((4, halo, halo, C), jnp.bfloat16),
            pltpu.VMEM((16, halo, halo, C), jnp.bfloat16),
        ],
        compiler_params=pltpu.CompilerParams(
            dimension_semantics=("parallel",),
            vmem_limit_bytes=_VMEM_LIMIT,
        ),
    )(x0p, yplanes, w9a, b3a, w1a, b1a, wct1, bct1, wc2, bc2,
      w9b, b3b, w1b, b1b, wct3, bct3, wct4, bct4)


def _planes_to_nchw(planes, B):
    P = planes.shape[3]
    y = jnp.transpose(planes, (0, 5, 3, 1, 4, 2))
    return y.reshape(B, 3, 8 * P, 8 * P)


def _nchw_to_padded_nhwc(x_nchw):
    x = jnp.transpose(x_nchw, (0, 2, 3, 1))
    x = jnp.pad(x, ((0, 0), (1, 1), (1, 1), (0, 0)))
    return x.astype(jnp.bfloat16)


def _nchw_to_planes(x_nchw):
    B, C, H, _ = x_nchw.shape
    t = jnp.transpose(x_nchw, (0, 2, 3, 1))
    t = t.reshape(B, H // 2, 2, H // 2, 2, C)
    t = jnp.transpose(t, (0, 2, 4, 1, 3, 5)).reshape(B, 4, H // 2, H // 2, C)
    t = jnp.pad(t, ((0, 0), (0, 0), (1, 1), (1, 1), (0, 0)))
    return t.astype(jnp.bfloat16)


def _pack_w9(w3):
    return jnp.concatenate(
        [w3[:, ky * 3 + kx] for kx in range(3) for ky in range(3)], axis=1)


def _pack_cat_w(wa, wb):
    parts = []
    for kx in range(3):
        for ky in range(3):
            t = ky * 3 + kx
            parts.append(wa[t])
            parts.append(wb[t])
    return jnp.concatenate(parts, axis=0)


def _pack_ct_w(wpar):
    C, Cop = wpar.shape[-2], wpar.shape[-1]
    z = jnp.zeros((C, Cop), wpar.dtype)
    rows = []
    for py in range(2):
        kblocks = []
        for c in range(3):
            for dy in range(2):
                left = wpar[2 * py, 2 * dy + c] if c <= 1 else z
                right = wpar[2 * py + 1, 2 * dy + c - 1] if c >= 1 else z
                kblocks.append(jnp.concatenate([left, right], axis=1))
        rows.append(jnp.concatenate(kblocks, axis=0))
    return jnp.stack(rows)


def _pack_ct_b(b):
    return jnp.concatenate([b, b], axis=1)


def kernel(x0, x1, r12_w3, r12_b3, r12_w1, r12_b1,
           r34_w3, r34_b3, r34_w1, r34_b1,
           ct1_w, ct1_b, ct3_w, ct3_b, ct4_w, ct4_b,
           c2_wa, c2_wb, c2_b):
    B = x0.shape[0]
    xp = _nchw_to_padded_nhwc(x0)
    yplanes = _nchw_to_planes(x1)
    packed = (
        _pack_w9(r12_w3), r12_b3, r12_w1, r12_b1,
        _pack_ct_w(ct1_w), _pack_ct_b(ct1_b),
        _pack_cat_w(c2_wa, c2_wb), c2_b,
        _pack_w9(r34_w3), r34_b3, r34_w1, r34_b1,
        _pack_ct_w(ct3_w), _pack_ct_b(ct3_b),
        _pack_ct_w(ct4_w), _pack_ct_b(ct4_b),
    )
    out = _decoder(xp, yplanes, packed)
    return _planes_to_nchw(out, B)


# speedup vs baseline: 1.1048x; 1.1048x over previous
import jax
import jax.numpy as jnp
from jax.experimental import pallas as pl
from jax.experimental.pallas import tpu as pltpu

_VMEM_LIMIT = 48 * 1024 * 1024


def _pc(i, d):
    return i % d, i // d + 1


def _halo_store(ref, interior, dtype=None):
    h, w, c = ref.shape
    ref[1:h - 1, 1:w - 1, :] = interior.astype(ref.dtype)


def _im2col9_single(x, H, W, C):
    cols = [x[:, kx:kx + W, :] for kx in range(3)]
    return jnp.concatenate(
        [cols[kx][ky:ky + H].reshape(H * W, C)
         for kx in range(3) for ky in range(3)],
        axis=-1)


def _plane_tap(planes, d, arow, acol, u, v, P, C):
    r, dr = _pc(arow + u - 1, d)
    c, dc = _pc(acol + v - 1, d)
    return planes[r * d + c][dr:dr + P, dc:dc + P, :].reshape(P * P, C)


def _im2col9_planes(planes, d, arow, acol, P, C):
    return jnp.concatenate(
        [_plane_tap(planes, d, arow, acol, u, v, P, C)
         for v in range(3) for u in range(3)],
        axis=-1)


def _ct_lhs(planes, d, arow, acol, py, P, C):
    pieces = []
    for c in range(3):
        cc, dc = _pc(acol + c - 1, d)
        for dy in range(2):
            rr, dr = _pc(arow + py + dy - 1, d)
            pieces.append(
                planes[rr * d + cc][dr:dr + P, dc:dc + P, :].reshape(
                    P * P, C))
    return jnp.concatenate(pieces, axis=-1)


def _res_block_single(xp, w9, b3, w1, b1, P, C):
    HW = P * P
    xr = jnp.maximum(xp, 0).astype(jnp.bfloat16)
    x9 = _im2col9_single(xr, P, P, C)
    acc = jnp.dot(x9, w9, preferred_element_type=jnp.float32)
    h = jnp.maximum(acc + b3, 0.0).astype(jnp.bfloat16)
    out = jnp.dot(h, w1, preferred_element_type=jnp.float32) + b1
    skip = xp[1:1 + P, 1:1 + P, :].reshape(HW, C).astype(jnp.float32)
    return out + skip


def _res_pair_planes(pin_refs, ptmp_refs, w9_ref, b3_ref, w1_ref, b1_ref,
                     P, C):
    HW = P * P
    n = len(pin_refs)
    d = 2 if n == 4 else 1
    for src, dst, blk in ((pin_refs, ptmp_refs, 0), (ptmp_refs, pin_refs, 1)):
        vals = [src[i][...] for i in range(n)]
        xr = [jnp.maximum(v, 0).astype(jnp.bfloat16) for v in vals]
        for a in range(d):
            for b in range(d):
                i = a * d + b
                x9 = _im2col9_planes(xr, d, a, b, P, C)
                acc = jnp.dot(x9, w9_ref[blk],
                              preferred_element_type=jnp.float32)
                h = jnp.maximum(acc + b3_ref[blk], 0.0).astype(jnp.bfloat16)
                out = jnp.dot(h, w1_ref[blk],
                              preferred_element_type=jnp.float32) + b1_ref[blk]
                skip = vals[i][1:1 + P, 1:1 + P, :].reshape(
                    HW, C).astype(jnp.float32)
                _halo_store(dst[i], (out + skip).reshape(P, P, C))


def _decoder_body(P, C):
    HW = P * P

    def body(x0_ref, y_ref,
             w9a_ref, b3a_ref, w1a_ref, b1a_ref,
             wct1_ref, bct1_ref,
             wc2_ref, bc2_ref,
             w9b_ref, b3b_ref, w1b_ref, b1b_ref,
             wct3_ref, bct3_ref,
             wct4_ref, bct4_ref,
             o_ref,
             m16_ref, pa_ref, pb_ref, pc_ref, p64_ref):
        pa = [pa_ref.at[i] for i in range(4)]
        pb = [pb_ref.at[i] for i in range(4)]
        pc = [pc_ref.at[i] for i in range(4)]
        p64 = [p64_ref.at[i] for i in range(16)]

        @pl.when(pl.program_id(0) == 0)
        def _init_borders():
            m16_ref[...] = jnp.zeros(m16_ref.shape, m16_ref.dtype)
            pa_ref[...] = jnp.zeros(pa_ref.shape, pa_ref.dtype)
            pb_ref[...] = jnp.zeros(pb_ref.shape, pb_ref.dtype)
            pc_ref[...] = jnp.zeros(pc_ref.shape, pc_ref.dtype)
            p64_ref[...] = jnp.zeros(p64_ref.shape, p64_ref.dtype)

        x1 = _res_block_single(x0_ref[0], w9a_ref[0], b3a_ref[0],
                               w1a_ref[0], b1a_ref[0], P, C)
        _halo_store(m16_ref, x1.reshape(P, P, C))
        x2 = _res_block_single(m16_ref[...], w9a_ref[1], b3a_ref[1],
                               w1a_ref[1], b1a_ref[1], P, C)
        _halo_store(m16_ref, x2.reshape(P, P, C))

        x = jnp.maximum(m16_ref[...], 0).astype(jnp.bfloat16)
        cols = [x[:, c:c + P, :] for c in range(3)]
        s = [[cols[c][r:r + P].reshape(HW, C) for r in range(3)]
             for c in range(3)]
        for py in range(2):
            lhs = jnp.concatenate(
                [s[c][py + dy] for c in range(3) for dy in range(2)], axis=-1)
            acc = jnp.dot(lhs, wct1_ref[py],
                          preferred_element_type=jnp.float32) + bct1_ref[...]
            acc = jnp.maximum(acc, 0.0)
            _halo_store(pa[2 * py], acc[:, :C].reshape(P, P, C))
            _halo_store(pa[2 * py + 1], acc[:, C:].reshape(P, P, C))

        avals = [pa_ref[i] for i in range(4)]
        yvals = [y_ref[0, i] for i in range(4)]
        for a in range(2):
            for b in range(2):
                pieces = []
                for v in range(3):
                    for u in range(3):
                        pieces.append(
                            _plane_tap(avals, 2, a, b, u, v, P, C))
                        pieces.append(
                            _plane_tap(yvals, 2, a, b, u, v, P, C))
                x18 = jnp.concatenate(pieces, axis=-1)
                out = jnp.dot(x18, wc2_ref[...],
                              preferred_element_type=jnp.float32)
                out = out + bc2_ref[...]
                _halo_store(pb[a * 2 + b], out.reshape(P, P, C))

        _res_pair_planes(pb, pc, w9b_ref, b3b_ref, w1b_ref, b1b_ref, P, C)

        xr = [jnp.maximum(pb_ref[i], 0).astype(jnp.bfloat16) for i in range(4)]
        for a in range(2):
            for b in range(2):
                for py in range(2):
                    lhs = _ct_lhs(xr, 2, a, b, py, P, C)
                    acc = jnp.dot(lhs, wct3_ref[py],
                                  preferred_element_type=jnp.float32)
                    acc = jnp.maximum(acc + bct3_ref[...], 0.0)
                    q = 2 * a + py
                    _halo_store(p64[q * 4 + 2 * b],
                                acc[:, :C].reshape(P, P, C))
                    _halo_store(p64[q * 4 + 2 * b + 1],
                                acc[:, C:].reshape(P, P, C))

        xv = [p64_ref[i] for i in range(16)]
        for q in range(4):
            for sidx in range(4):
                for py in range(2):
                    lhs = _ct_lhs(xv, 4, q, sidx, py, P, C)
                    acc = jnp.dot(lhs, wct4_ref[py],
                                  preferred_element_type=jnp.float32)
                    acc = acc + bct4_ref[...]
                    r8 = 2 * q + py
                    o_ref[0, r8, 2 * sidx] = acc[:, :3].reshape(P, P, 3)
                    o_ref[0, r8, 2 * sidx + 1] = acc[:, C:C + 3].reshape(
                        P, P, 3)

    return body


def _const_spec(*shape):
    nz = len(shape)
    return pl.BlockSpec(shape, lambda b, _n=nz: (0,) * _n)


def _decoder(x0p, yplanes, packed):
    B = x0p.shape[0]
    C = x0p.shape[-1]
    P = x0p.shape[1] - 2
    (w9a, b3a, w1a, b1a, wct1, bct1, wc2, bc2,
     w9b, b3b, w1b, b1b, wct3, bct3, wct4, bct4) = packed
    halo = P + 2
    return pl.pallas_call(
        _decoder_body(P, C),
        out_shape=jax.ShapeDtypeStruct((B, 8, 8, P, P, 3), jnp.float32),
        grid=(B,),
        in_specs=[
            pl.BlockSpec((1, halo, halo, C), lambda b: (b, 0, 0, 0)),
            pl.BlockSpec((1, 4, halo, halo, C), lambda b: (b, 0, 0, 0, 0)),
            _const_spec(2, 9 * C, C), _const_spec(2, 1, C),
            _const_spec(2, C, C), _const_spec(2, 1, C),
            _const_spec(2, 6 * C, 2 * C), _const_spec(1, 2 * C),
            _const_spec(18 * C, C), _const_spec(1, C),
            _const_spec(2, 9 * C, C), _const_spec(2, 1, C),
            _const_spec(2, C, C), _const_spec(2, 1, C),
            _const_spec(2, 6 * C, 2 * C), _const_spec(1, 2 * C),
            _const_spec(2, 6 * C, 2 * C), _const_spec(1, 2 * C),
        ],
        out_specs=pl.BlockSpec((1, 8, 8, P, P, 3),
                               lambda b: (b, 0, 0, 0, 0, 0)),
        scratch_shapes=[
            pltpu.VMEM((halo, halo, C), jnp.bfloat16),
            pltpu.VMEM((4, halo, halo, C), jnp.bfloat16),
            pltpu.VMEM((4, halo, halo, C), jnp.bfloat16),
            pltpu.VMEM((4, halo, halo, C), jnp.bfloat16),
            pltpu.VMEM((16, halo, halo, C), jnp.bfloat16),
        ],
        compiler_params=pltpu.CompilerParams(
            dimension_semantics=("parallel",),
            vmem_limit_bytes=_VMEM_LIMIT,
        ),
    )(x0p, yplanes, w9a, b3a, w1a, b1a, wct1, bct1, wc2, bc2,
      w9b, b3b, w1b, b1b, wct3, bct3, wct4, bct4)


def _planes_to_nchw(planes, B):
    P = planes.shape[3]
    y = jnp.transpose(planes, (0, 5, 3, 1, 4, 2))
    return y.reshape(B, 3, 8 * P, 8 * P)


def _nchw_to_padded_nhwc(x_nchw):
    x = jnp.transpose(x_nchw, (0, 2, 3, 1))
    x = jnp.pad(x, ((0, 0), (1, 1), (1, 1), (0, 0)))
    return x.astype(jnp.bfloat16)


def _nchw_to_planes(x_nchw):
    B, C, H, _ = x_nchw.shape
    t = jnp.transpose(x_nchw, (0, 2, 3, 1))
    t = t.reshape(B, H // 2, 2, H // 2, 2, C)
    t = jnp.transpose(t, (0, 2, 4, 1, 3, 5)).reshape(B, 4, H // 2, H // 2, C)
    t = jnp.pad(t, ((0, 0), (0, 0), (1, 1), (1, 1), (0, 0)))
    return t.astype(jnp.bfloat16)


def _pack_w9(w3):
    return jnp.concatenate(
        [w3[:, ky * 3 + kx] for kx in range(3) for ky in range(3)], axis=1)


def _pack_cat_w(wa, wb):
    parts = []
    for kx in range(3):
        for ky in range(3):
            t = ky * 3 + kx
            parts.append(wa[t])
            parts.append(wb[t])
    return jnp.concatenate(parts, axis=0)


def _pack_ct_w(wpar):
    C, Cop = wpar.shape[-2], wpar.shape[-1]
    z = jnp.zeros((C, Cop), wpar.dtype)
    rows = []
    for py in range(2):
        kblocks = []
        for c in range(3):
            for dy in range(2):
                left = wpar[2 * py, 2 * dy + c] if c <= 1 else z
                right = wpar[2 * py + 1, 2 * dy + c - 1] if c >= 1 else z
                kblocks.append(jnp.concatenate([left, right], axis=1))
        rows.append(jnp.concatenate(kblocks, axis=0))
    return jnp.stack(rows)


def _pack_ct_b(b):
    return jnp.concatenate([b, b], axis=1)


def kernel(x0, x1, r12_w3, r12_b3, r12_w1, r12_b1,
           r34_w3, r34_b3, r34_w1, r34_b1,
           ct1_w, ct1_b, ct3_w, ct3_b, ct4_w, ct4_b,
           c2_wa, c2_wb, c2_b):
    B = x0.shape[0]
    xp = _nchw_to_padded_nhwc(x0)
    yplanes = _nchw_to_planes(x1)
    packed = (
        _pack_w9(r12_w3), r12_b3, r12_w1, r12_b1,
        _pack_ct_w(ct1_w), _pack_ct_b(ct1_b),
        _pack_cat_w(c2_wa, c2_wb), c2_b,
        _pack_w9(r34_w3), r34_b3, r34_w1, r34_b1,
        _pack_ct_w(ct3_w), _pack_ct_b(ct3_b),
        _pack_ct_w(ct4_w), _pack_ct_b(ct4_b),
    )
    out = _decoder(xp, yplanes, packed)
    return _planes_to_nchw(out, B)


# speedup vs baseline: 1.1484x; 1.0395x over previous
import jax
import jax.numpy as jnp
from jax.experimental import pallas as pl
from jax.experimental.pallas import tpu as pltpu

_VMEM_LIMIT = 48 * 1024 * 1024


def _pc(i, d):
    return i % d, i // d + 1


def _halo_store(ref, interior, dtype=None):
    h, w, c = ref.shape
    ref[1:h - 1, 1:w - 1, :] = interior.astype(ref.dtype)


def _im2col9_single(x, H, W, C):
    cols = [x[:, kx:kx + W, :] for kx in range(3)]
    return jnp.concatenate(
        [cols[kx][ky:ky + H].reshape(H * W, C)
         for kx in range(3) for ky in range(3)],
        axis=-1)


def _plane_tap(planes, d, arow, acol, u, v, P, C):
    r, dr = _pc(arow + u - 1, d)
    c, dc = _pc(acol + v - 1, d)
    return planes[r * d + c][dr:dr + P, dc:dc + P, :].reshape(P * P, C)


def _im2col9_planes(planes, d, arow, acol, P, C):
    return jnp.concatenate(
        [_plane_tap(planes, d, arow, acol, u, v, P, C)
         for v in range(3) for u in range(3)],
        axis=-1)


def _ct_lhs(planes, d, arow, acol, py, P, C):
    pieces = []
    for c in range(3):
        cc, dc = _pc(acol + c - 1, d)
        for dy in range(2):
            rr, dr = _pc(arow + py + dy - 1, d)
            pieces.append(
                planes[rr * d + cc][dr:dr + P, dc:dc + P, :].reshape(
                    P * P, C))
    return jnp.concatenate(pieces, axis=-1)


def _res_block_single(xp, w9, b3, w1, b1, P, C):
    HW = P * P
    xr = jnp.maximum(xp, 0).astype(jnp.bfloat16)
    x9 = _im2col9_single(xr, P, P, C)
    acc = jnp.dot(x9, w9, preferred_element_type=jnp.float32)
    h = jnp.maximum(acc + b3, 0.0).astype(jnp.bfloat16)
    out = jnp.dot(h, w1, preferred_element_type=jnp.float32) + b1
    skip = xp[1:1 + P, 1:1 + P, :].reshape(HW, C).astype(jnp.float32)
    return out + skip


def _pair_tap(planes, arow, sc, u, P, C):
    r, dr = _pc(arow + u - 1, 2)
    c, dc = _pc(sc, 2)
    return planes[r * 2 + c][dr:dr + P, dc:dc + P, :].reshape(P * P, C)


def _res_pair_planes(pin_refs, ptmp_refs, w9_ref, b3_ref, w1_ref, b1_ref,
                     P, C):
    HW = P * P
    for src, dst, blk in ((pin_refs, ptmp_refs, 0), (ptmp_refs, pin_refs, 1)):
        vals = [src[i][...] for i in range(4)]
        xr = [jnp.maximum(v, 0).astype(jnp.bfloat16) for v in vals]
        for a in range(2):
            x12 = jnp.concatenate(
                [_pair_tap(xr, a, sc, u, P, C)
                 for sc in (-1, 0, 1, 2) for u in range(3)], axis=-1)
            acc = jnp.dot(x12, w9_ref[blk],
                          preferred_element_type=jnp.float32)
            h = jnp.maximum(acc + b3_ref[blk], 0.0).astype(jnp.bfloat16)
            out = jnp.dot(h, w1_ref[blk],
                          preferred_element_type=jnp.float32) + b1_ref[blk]
            skip = jnp.concatenate(
                [vals[2 * a + b][1:1 + P, 1:1 + P, :].reshape(HW, C)
                 for b in range(2)], axis=-1).astype(jnp.float32)
            res = out + skip
            _halo_store(dst[2 * a], res[:, :C].reshape(P, P, C))
            _halo_store(dst[2 * a + 1], res[:, C:].reshape(P, P, C))


def _decoder_body(P, C):
    HW = P * P

    def body(x0_ref, y_ref,
             w9a_ref, b3a_ref, w1a_ref, b1a_ref,
             wct1_ref, bct1_ref,
             wc2_ref, bc2_ref,
             w9b_ref, b3b_ref, w1b_ref, b1b_ref,
             wct3_ref, bct3_ref,
             wct4_ref, bct4_ref,
             o_ref,
             m16_ref, pa_ref, pb_ref, pc_ref, p64_ref):
        pa = [pa_ref.at[i] for i in range(4)]
        pb = [pb_ref.at[i] for i in range(4)]
        pc = [pc_ref.at[i] for i in range(4)]
        p64 = [p64_ref.at[i] for i in range(16)]

        @pl.when(pl.program_id(0) == 0)
        def _init_borders():
            m16_ref[...] = jnp.zeros(m16_ref.shape, m16_ref.dtype)
            pa_ref[...] = jnp.zeros(pa_ref.shape, pa_ref.dtype)
            pb_ref[...] = jnp.zeros(pb_ref.shape, pb_ref.dtype)
            pc_ref[...] = jnp.zeros(pc_ref.shape, pc_ref.dtype)
            p64_ref[...] = jnp.zeros(p64_ref.shape, p64_ref.dtype)

        x1 = _res_block_single(x0_ref[0], w9a_ref[0], b3a_ref[0],
                               w1a_ref[0], b1a_ref[0], P, C)
        _halo_store(m16_ref, x1.reshape(P, P, C))
        x2 = _res_block_single(m16_ref[...], w9a_ref[1], b3a_ref[1],
                               w1a_ref[1], b1a_ref[1], P, C)
        _halo_store(m16_ref, x2.reshape(P, P, C))

        x = jnp.maximum(m16_ref[...], 0).astype(jnp.bfloat16)
        cols = [x[:, c:c + P, :] for c in range(3)]
        s = [[cols[c][r:r + P].reshape(HW, C) for r in range(3)]
             for c in range(3)]
        for py in range(2):
            lhs = jnp.concatenate(
                [s[c][py + dy] for c in range(3) for dy in range(2)], axis=-1)
            acc = jnp.dot(lhs, wct1_ref[py],
                          preferred_element_type=jnp.float32) + bct1_ref[...]
            acc = jnp.maximum(acc, 0.0)
            _halo_store(pa[2 * py], acc[:, :C].reshape(P, P, C))
            _halo_store(pa[2 * py + 1], acc[:, C:].reshape(P, P, C))

        avals = [pa_ref[i] for i in range(4)]
        yvals = [y_ref[0, i] for i in range(4)]
        for a in range(2):
            pieces = []
            for sc in (-1, 0, 1, 2):
                for u in range(3):
                    pieces.append(_pair_tap(avals, a, sc, u, P, C))
                    pieces.append(_pair_tap(yvals, a, sc, u, P, C))
            x24 = jnp.concatenate(pieces, axis=-1)
            out = jnp.dot(x24, wc2_ref[...],
                          preferred_element_type=jnp.float32) + bc2_ref[...]
            _halo_store(pb[2 * a], out[:, :C].reshape(P, P, C))
            _halo_store(pb[2 * a + 1], out[:, C:].reshape(P, P, C))

        _res_pair_planes(pb, pc, w9b_ref, b3b_ref, w1b_ref, b1b_ref, P, C)

        xr = [jnp.maximum(pb_ref[i], 0).astype(jnp.bfloat16) for i in range(4)]
        for a in range(2):
            for b in range(2):
                for py in range(2):
                    lhs = _ct_lhs(xr, 2, a, b, py, P, C)
                    acc = jnp.dot(lhs, wct3_ref[py],
                                  preferred_element_type=jnp.float32)
                    acc = jnp.maximum(acc + bct3_ref[...], 0.0)
                    q = 2 * a + py
                    _halo_store(p64[q * 4 + 2 * b],
                                acc[:, :C].reshape(P, P, C))
                    _halo_store(p64[q * 4 + 2 * b + 1],
                                acc[:, C:].reshape(P, P, C))

        xv = [p64_ref[i] for i in range(16)]
        for q in range(4):
            for sidx in range(4):
                for py in range(2):
                    lhs = _ct_lhs(xv, 4, q, sidx, py, P, C)
                    acc = jnp.dot(lhs, wct4_ref[py],
                                  preferred_element_type=jnp.float32)
                    acc = acc + bct4_ref[...]
                    r8 = 2 * q + py
                    o_ref[0, r8, 2 * sidx] = acc[:, :3].reshape(P, P, 3)
                    o_ref[0, r8, 2 * sidx + 1] = acc[:, C:C + 3].reshape(
                        P, P, 3)

    return body


def _const_spec(*shape):
    nz = len(shape)
    return pl.BlockSpec(shape, lambda b, _n=nz: (0,) * _n)


def _decoder(x0p, yplanes, packed):
    B = x0p.shape[0]
    C = x0p.shape[-1]
    P = x0p.shape[1] - 2
    (w9a, b3a, w1a, b1a, wct1, bct1, wc2, bc2,
     w9b, b3b, w1b, b1b, wct3, bct3, wct4, bct4) = packed
    halo = P + 2
    return pl.pallas_call(
        _decoder_body(P, C),
        out_shape=jax.ShapeDtypeStruct((B, 8, 8, P, P, 3), jnp.float32),
        grid=(B,),
        in_specs=[
            pl.BlockSpec((1, halo, halo, C), lambda b: (b, 0, 0, 0)),
            pl.BlockSpec((1, 4, halo, halo, C), lambda b: (b, 0, 0, 0, 0)),
            _const_spec(2, 9 * C, C), _const_spec(2, 1, C),
            _const_spec(2, C, C), _const_spec(2, 1, C),
            _const_spec(2, 6 * C, 2 * C), _const_spec(1, 2 * C),
            _const_spec(24 * C, 2 * C), _const_spec(1, 2 * C),
            _const_spec(2, 12 * C, 2 * C), _const_spec(2, 1, 2 * C),
            _const_spec(2, 2 * C, 2 * C), _const_spec(2, 1, 2 * C),
            _const_spec(2, 6 * C, 2 * C), _const_spec(1, 2 * C),
            _const_spec(2, 6 * C, 2 * C), _const_spec(1, 2 * C),
        ],
        out_specs=pl.BlockSpec((1, 8, 8, P, P, 3),
                               lambda b: (b, 0, 0, 0, 0, 0)),
        scratch_shapes=[
            pltpu.VMEM((halo, halo, C), jnp.bfloat16),
            pltpu.VMEM((4, halo, halo, C), jnp.bfloat16),
            pltpu.VMEM((4, halo, halo, C), jnp.bfloat16),
            pltpu.VMEM((4, halo, halo, C), jnp.bfloat16),
            pltpu.VMEM((16, halo, halo, C), jnp.bfloat16),
        ],
        compiler_params=pltpu.CompilerParams(
            dimension_semantics=("parallel",),
            vmem_limit_bytes=_VMEM_LIMIT,
        ),
    )(x0p, yplanes, w9a, b3a, w1a, b1a, wct1, bct1, wc2, bc2,
      w9b, b3b, w1b, b1b, wct3, bct3, wct4, bct4)


def _planes_to_nchw(planes, B):
    P = planes.shape[3]
    y = jnp.transpose(planes, (0, 5, 3, 1, 4, 2))
    return y.reshape(B, 3, 8 * P, 8 * P)


def _nchw_to_padded_nhwc(x_nchw):
    x = jnp.transpose(x_nchw, (0, 2, 3, 1))
    x = jnp.pad(x, ((0, 0), (1, 1), (1, 1), (0, 0)))
    return x.astype(jnp.bfloat16)


def _nchw_to_planes(x_nchw):
    B, C, H, _ = x_nchw.shape
    t = jnp.transpose(x_nchw, (0, 2, 3, 1))
    t = t.reshape(B, H // 2, 2, H // 2, 2, C)
    t = jnp.transpose(t, (0, 2, 4, 1, 3, 5)).reshape(B, 4, H // 2, H // 2, C)
    t = jnp.pad(t, ((0, 0), (0, 0), (1, 1), (1, 1), (0, 0)))
    return t.astype(jnp.bfloat16)


def _pack_w9(w3):
    return jnp.concatenate(
        [w3[:, ky * 3 + kx] for kx in range(3) for ky in range(3)], axis=1)


def _pair_w_block(w3, sc, u):
    C = w3.shape[-2]
    z = jnp.zeros((C, C), w3.dtype)
    left = w3[u * 3 + sc + 1] if 0 <= sc + 1 <= 2 else z
    right = w3[u * 3 + sc] if 0 <= sc <= 2 else z
    return jnp.concatenate([left, right], axis=1)


def _pack_w3_pair(w3):
    rows = []
    for blk in range(2):
        rows.append(jnp.concatenate(
            [_pair_w_block(w3[blk], sc, u)
             for sc in (-1, 0, 1, 2) for u in range(3)], axis=0))
    return jnp.stack(rows)


def _pack_w1_bd(w1):
    C = w1.shape[-1]
    z = jnp.zeros((C, C), w1.dtype)
    return jnp.stack([
        jnp.concatenate([jnp.concatenate([w1[b], z], axis=1),
                         jnp.concatenate([z, w1[b]], axis=1)], axis=0)
        for b in range(2)])


def _pack_cat_w_pair(wa, wb):
    parts = []
    for sc in (-1, 0, 1, 2):
        for u in range(3):
            parts.append(_pair_w_block(wa, sc, u))
            parts.append(_pair_w_block(wb, sc, u))
    return jnp.concatenate(parts, axis=0)


def _tile2(b):
    return jnp.concatenate([b, b], axis=-1)


def _pack_ct_w(wpar):
    C, Cop = wpar.shape[-2], wpar.shape[-1]
    z = jnp.zeros((C, Cop), wpar.dtype)
    rows = []
    for py in range(2):
        kblocks = []
        for c in range(3):
            for dy in range(2):
                left = wpar[2 * py, 2 * dy + c] if c <= 1 else z
                right = wpar[2 * py + 1, 2 * dy + c - 1] if c >= 1 else z
                kblocks.append(jnp.concatenate([left, right], axis=1))
        rows.append(jnp.concatenate(kblocks, axis=0))
    return jnp.stack(rows)


def _pack_ct_b(b):
    return jnp.concatenate([b, b], axis=1)


def kernel(x0, x1, r12_w3, r12_b3, r12_w1, r12_b1,
           r34_w3, r34_b3, r34_w1, r34_b1,
           ct1_w, ct1_b, ct3_w, ct3_b, ct4_w, ct4_b,
           c2_wa, c2_wb, c2_b):
    B = x0.shape[0]
    xp = _nchw_to_padded_nhwc(x0)
    yplanes = _nchw_to_planes(x1)
    packed = (
        _pack_w9(r12_w3), r12_b3, r12_w1, r12_b1,
        _pack_ct_w(ct1_w), _pack_ct_b(ct1_b),
        _pack_cat_w_pair(c2_wa, c2_wb), _tile2(c2_b),
        _pack_w3_pair(r34_w3), _tile2(r34_b3),
        _pack_w1_bd(r34_w1), _tile2(r34_b1),
        _pack_ct_w(ct3_w), _pack_ct_b(ct3_b),
        _pack_ct_w(ct4_w), _pack_ct_b(ct4_b),
    )
    out = _decoder(xp, yplanes, packed)
    return _planes_to_nchw(out, B)
